# Initial kernel scaffold; baseline (speedup 1.0000x reference)
import jax
import jax.numpy as jnp
from jax.experimental import pallas as pl


def kernel(x, edge_index, edge_attr, batch, W1_0, b1_0, W2_0, b2_0, W1_1, b1_1, W2_1, b2_1, W1_2, b1_2, W2_2, b2_2, W_fc, b_fc):
    raise NotImplementedError("write your pallas kernel here")


# speedup vs baseline: 5.4562x; 5.4562x over previous
import functools

import jax
import jax.numpy as jnp
from jax import lax
from jax.experimental import pallas as pl
from jax.experimental.pallas import tpu as pltpu
from jax.experimental.pallas import tpu_sc as plsc

_N = 10000
_E = 320000
_G = 64
_NTILES = 16
_K = 80
_NP = 10240
_RPT = _NP // _NTILES
_NGRP = 5


@functools.cache
def _make_sc_agg(mode):
    nblk = _NTILES if mode == "col" else 2 * _NTILES
    grows = _E // (nblk * _K * _NGRP)
    mesh = plsc.VectorSubcoreMesh(core_axis_name="c", subcore_axis_name="s")

    @functools.partial(
        pl.kernel,
        out_type=jax.ShapeDtypeStruct((2, _NP, 128), jnp.float32),
        mesh=mesh,
        scratch_types=[
            pltpu.VMEM((grows, _K), jnp.int32),
            pltpu.VMEM((grows, _K), jnp.int32),
            pltpu.VMEM((_K,), jnp.int32),
            pltpu.VMEM((_K, 128), jnp.float32),
            pltpu.VMEM_SHARED((_NP, 128), jnp.float32),
            pltpu.SemaphoreType.DMA,
        ],
    )
    def k(h2, src_r, dst_r, out, src_v, dst_v, idx_v, buf_v, acc, sem):
        c = lax.axis_index("c")
        s = lax.axis_index("s")
        blk = s if mode == "col" else c * _NTILES + s

        zeros16 = jnp.zeros((16,), jnp.float32)

        @pl.loop(0, _K)
        def _zero_row(r):
            for i in range(8):
                buf_v[r, pl.ds(i * 16, 16)] = zeros16

        r0 = s * _RPT

        @pl.loop(0, _RPT // _K)
        def _zero_acc(t):
            pltpu.sync_copy(buf_v, acc.at[pl.ds(r0 + t * _K, _K)])

        plsc.subcore_barrier()

        @pl.loop(0, _NGRP)
        def _group(g):
            pltpu.sync_copy(src_r.at[blk, g], src_v)
            pltpu.sync_copy(dst_r.at[blk, g], dst_v)

            @pl.loop(0, grows)
            def _chunk(j):
                if mode == "col":
                    for i in range(_K // 16):
                        sl = pl.ds(i * 16, 16)
                        idx_v[sl] = src_v[j, sl] * 2 + c
                    pltpu.async_copy(h2.at[idx_v], buf_v, sem).wait()
                else:
                    pltpu.async_copy(h2.at[src_v.at[j]], buf_v, sem).wait()
                pltpu.sync_copy(buf_v, acc.at[dst_v.at[j]], add=True)

        plsc.subcore_barrier()
        pltpu.sync_copy(acc.at[pl.ds(r0, _RPT)], out.at[c, pl.ds(r0, _RPT)])

    return k


def _mlp_body2(h_ref, a_ref, b_ref2, w1_ref, b1_ref, w2_ref, b2_ref, o_ref):
    _mlp_core(h_ref[...] + a_ref[...] + b_ref2[...],
              w1_ref, b1_ref, w2_ref, b2_ref, o_ref)


def _mlp_body1(h_ref, a_ref, w1_ref, b1_ref, w2_ref, b2_ref, o_ref):
    _mlp_core(h_ref[...] + a_ref[...], w1_ref, b1_ref, w2_ref, b2_ref, o_ref)


def _mlp_core(z, w1_ref, b1_ref, w2_ref, b2_ref, o_ref):
    z = jnp.dot(z, w1_ref[...], preferred_element_type=jnp.float32) + b1_ref[...]
    z = jnp.maximum(z, 0.0)
    z = jnp.dot(z, w2_ref[...], preferred_element_type=jnp.float32) + b2_ref[...]
    o_ref[...] = jnp.maximum(z, 0.0)


def _tc_mlp(h, aggs, w1, b1, w2, b2, block=1000):
    n, d = h.shape
    dh = w1.shape[1]
    row_spec = pl.BlockSpec((block, d), lambda i: (i, 0))
    return pl.pallas_call(
        _mlp_body1 if len(aggs) == 1 else _mlp_body2,
        grid=(n // block,),
        in_specs=[row_spec] * (1 + len(aggs)) + [
            pl.BlockSpec((d, dh), lambda i: (0, 0)),
            pl.BlockSpec((1, dh), lambda i: (0, 0)),
            pl.BlockSpec((dh, dh), lambda i: (0, 0)),
            pl.BlockSpec((1, dh), lambda i: (0, 0)),
        ],
        out_specs=pl.BlockSpec((block, dh), lambda i: (i, 0)),
        out_shape=jax.ShapeDtypeStruct((n, dh), jnp.float32),
    )(h, *aggs, w1, b1.reshape(1, -1), w2, b2.reshape(1, -1))


def _pool_body(b_ref, h_ref, wfc_ref, bfc_ref, o_ref, sums, counts):
    i = pl.program_id(0)
    nb = pl.num_programs(0)

    @pl.when(i == 0)
    def _():
        sums[...] = jnp.zeros_like(sums)
        counts[...] = jnp.zeros_like(counts)

    bvec = b_ref[0]
    gids = lax.broadcasted_iota(jnp.int32, (_G, bvec.shape[1]), 0)
    m = (bvec == gids).astype(jnp.float32)
    sums[...] += jnp.dot(m, h_ref[...], preferred_element_type=jnp.float32)
    counts[...] += jnp.sum(m, axis=1, keepdims=True)

    @pl.when(i == nb - 1)
    def _():
        pooled = sums[...] / jnp.maximum(counts[...], 1.0)
        o_ref[...] = (
            jnp.dot(pooled, wfc_ref[...], preferred_element_type=jnp.float32)
            + bfc_ref[...]
        )


def _tc_pool_fc(batch, h, w_fc, b_fc, block=1000):
    n, d = h.shape
    dout = w_fc.shape[1]
    nb = n // block
    batch_r = batch.reshape(nb, 1, block)
    return pl.pallas_call(
        _pool_body,
        grid=(nb,),
        in_specs=[
            pl.BlockSpec((1, 1, block), lambda i: (i, 0, 0)),
            pl.BlockSpec((block, d), lambda i: (i, 0)),
            pl.BlockSpec((d, dout), lambda i: (0, 0)),
            pl.BlockSpec((1, dout), lambda i: (0, 0)),
        ],
        out_specs=pl.BlockSpec((_G, dout), lambda i: (0, 0)),
        out_shape=jax.ShapeDtypeStruct((_G, dout), jnp.float32),
        scratch_shapes=[
            pltpu.VMEM((_G, d), jnp.float32),
            pltpu.VMEM((_G, 1), jnp.float32),
        ],
    )(batch_r, h, w_fc, b_fc.reshape(1, -1))


def kernel(x, edge_index, edge_attr, batch,
           W1_0, b1_0, W2_0, b2_0,
           W1_1, b1_1, W2_1, b2_1,
           W1_2, b1_2, W2_2, b2_2,
           W_fc, b_fc):
    del edge_attr
    src = edge_index[0]
    dst = edge_index[1]
    src_e = src.reshape(2 * _NTILES, _NGRP, -1, _K)
    dst_e = dst.reshape(2 * _NTILES, _NGRP, -1, _K)
    src_c = src.reshape(_NTILES, _NGRP, -1, _K)
    dst_c = dst.reshape(_NTILES, _NGRP, -1, _K)

    agg2 = _make_sc_agg("edge")(x, src_e, dst_e)
    h = _tc_mlp(x, [agg2[0, :_N], agg2[1, :_N]], W1_0, b1_0, W2_0, b2_0)

    for (w1, b1, w2, b2) in [(W1_1, b1_1, W2_1, b2_1), (W1_2, b1_2, W2_2, b2_2)]:
        h2 = h.reshape(2 * _N, 128)
        agg2 = _make_sc_agg("col")(h2, src_c, dst_c)
        agg = agg2[:, :_N].transpose(1, 0, 2).reshape(_N, 256)
        h = _tc_mlp(h, [agg], w1, b1, w2, b2)

    return _tc_pool_fc(batch, h, W_fc, b_fc)


# speedup vs baseline: 6.7188x; 1.2314x over previous
import functools

import jax
import jax.numpy as jnp
from jax import lax
from jax.experimental import pallas as pl
from jax.experimental.pallas import tpu as pltpu
from jax.experimental.pallas import tpu_sc as plsc

_N = 10000
_E = 320000
_G = 64
_NTILES = 16
_K = 80
_NP = 10240
_RPT = _NP // _NTILES


@functools.cache
def _make_sc_agg(mode):
    nblk = _NTILES if mode == "col" else 2 * _NTILES
    ngrp = 25
    grows = _E // (nblk * _K * ngrp)
    mesh = plsc.VectorSubcoreMesh(core_axis_name="c", subcore_axis_name="s")

    @functools.partial(
        pl.kernel,
        out_type=jax.ShapeDtypeStruct((2, _NP, 128), jnp.float32),
        mesh=mesh,
        scratch_types=[
            pltpu.VMEM((grows, _K), jnp.int32),
            pltpu.VMEM((grows, _K), jnp.int32),
            pltpu.VMEM((2, _K), jnp.int32),
            pltpu.VMEM((2, _K, 128), jnp.float32),
            pltpu.VMEM_SHARED((_NP, 128), jnp.float32),
            pltpu.SemaphoreType.DMA,
            pltpu.SemaphoreType.DMA,
        ],
    )
    def k(h2, src_r, dst_r, out, src_v, dst_v, gv, buf, acc, gsem, ssem):
        c = lax.axis_index("c")
        s = lax.axis_index("s")
        blk = s if mode == "col" else c * _NTILES + s

        zeros16 = jnp.zeros((16,), jnp.float32)

        @pl.loop(0, _K)
        def _zero_row(r):
            for i in range(8):
                buf[0, r, pl.ds(i * 16, 16)] = zeros16

        r0 = s * _RPT

        @pl.loop(0, _RPT // _K)
        def _zero_acc(t):
            pltpu.sync_copy(buf.at[0], acc.at[pl.ds(r0 + t * _K, _K)])

        plsc.subcore_barrier()

        def fire_gather(j, b):
            if mode == "col":
                for i in range(_K // 16):
                    sl = pl.ds(i * 16, 16)
                    gv[b, sl] = src_v[j, sl] * 2 + c
                return pltpu.async_copy(h2.at[gv.at[b]], buf.at[b], gsem)
            return pltpu.async_copy(h2.at[src_v.at[j]], buf.at[b], gsem)

        def fire_scatter(j, b):
            return pltpu.async_copy(buf.at[b], acc.at[dst_v.at[j]], ssem,
                                    add=True)

        @pl.loop(0, ngrp)
        def _group(g):
            pltpu.sync_copy(src_r.at[blk, g], src_v)
            pltpu.sync_copy(dst_r.at[blk, g], dst_v)
            gathers = [None, None]
            scatters = [None, None]
            gathers[0] = fire_gather(0, 0)
            for j in range(grows):
                b = j % 2
                bn = 1 - b
                gathers[b].wait()
                scatters[b] = fire_scatter(j, b)
                if j + 1 < grows:
                    if scatters[bn] is not None:
                        scatters[bn].wait()
                    gathers[bn] = fire_gather(j + 1, bn)
            scatters[(grows - 1) % 2].wait()

        plsc.subcore_barrier()
        pltpu.sync_copy(acc.at[pl.ds(r0, _RPT)], out.at[c, pl.ds(r0, _RPT)])

    return k


def _mlp_body_edge(h_ref, a_ref, w1_ref, b1_ref, w2_ref, b2_ref, o_ref):
    _mlp_core(h_ref[...] + a_ref[0] + a_ref[1],
              w1_ref, b1_ref, w2_ref, b2_ref, o_ref)


def _mlp_body_col(h_ref, a_ref, w1_ref, b1_ref, w2_ref, b2_ref, o_ref):
    agg = jnp.concatenate([a_ref[0], a_ref[1]], axis=1)
    _mlp_core(h_ref[...] + agg, w1_ref, b1_ref, w2_ref, b2_ref, o_ref)


def _mlp_core(z, w1_ref, b1_ref, w2_ref, b2_ref, o_ref):
    z = jnp.dot(z, w1_ref[...], preferred_element_type=jnp.float32) + b1_ref[...]
    z = jnp.maximum(z, 0.0)
    z = jnp.dot(z, w2_ref[...], preferred_element_type=jnp.float32) + b2_ref[...]
    o_ref[...] = jnp.maximum(z, 0.0)


def _tc_mlp(h, agg2, mode, w1, b1, w2, b2, block=1000):
    n, d = h.shape
    dh = w1.shape[1]
    return pl.pallas_call(
        _mlp_body_col if mode == "col" else _mlp_body_edge,
        grid=(n // block,),
        in_specs=[
            pl.BlockSpec((block, d), lambda i: (i, 0)),
            pl.BlockSpec((2, block, 128), lambda i: (0, i, 0)),
            pl.BlockSpec((d, dh), lambda i: (0, 0)),
            pl.BlockSpec((1, dh), lambda i: (0, 0)),
            pl.BlockSpec((dh, dh), lambda i: (0, 0)),
            pl.BlockSpec((1, dh), lambda i: (0, 0)),
        ],
        out_specs=pl.BlockSpec((block, dh), lambda i: (i, 0)),
        out_shape=jax.ShapeDtypeStruct((n, dh), jnp.float32),
    )(h, agg2, w1, b1.reshape(1, -1), w2, b2.reshape(1, -1))


def _pool_body(b_ref, h_ref, wfc_ref, bfc_ref, o_ref, sums, counts):
    i = pl.program_id(0)
    nb = pl.num_programs(0)

    @pl.when(i == 0)
    def _():
        sums[...] = jnp.zeros_like(sums)
        counts[...] = jnp.zeros_like(counts)

    bvec = b_ref[0]
    gids = lax.broadcasted_iota(jnp.int32, (_G, bvec.shape[1]), 0)
    m = (bvec == gids).astype(jnp.float32)
    sums[...] += jnp.dot(m, h_ref[...], preferred_element_type=jnp.float32)
    counts[...] += jnp.sum(m, axis=1, keepdims=True)

    @pl.when(i == nb - 1)
    def _():
        pooled = sums[...] / jnp.maximum(counts[...], 1.0)
        o_ref[...] = (
            jnp.dot(pooled, wfc_ref[...], preferred_element_type=jnp.float32)
            + bfc_ref[...]
        )


def _tc_pool_fc(batch, h, w_fc, b_fc, block=1000):
    n, d = h.shape
    dout = w_fc.shape[1]
    nb = n // block
    batch_r = batch.reshape(nb, 1, block)
    return pl.pallas_call(
        _pool_body,
        grid=(nb,),
        in_specs=[
            pl.BlockSpec((1, 1, block), lambda i: (i, 0, 0)),
            pl.BlockSpec((block, d), lambda i: (i, 0)),
            pl.BlockSpec((d, dout), lambda i: (0, 0)),
            pl.BlockSpec((1, dout), lambda i: (0, 0)),
        ],
        out_specs=pl.BlockSpec((_G, dout), lambda i: (0, 0)),
        out_shape=jax.ShapeDtypeStruct((_G, dout), jnp.float32),
        scratch_shapes=[
            pltpu.VMEM((_G, d), jnp.float32),
            pltpu.VMEM((_G, 1), jnp.float32),
        ],
    )(batch_r, h, w_fc, b_fc.reshape(1, -1))


def kernel(x, edge_index, edge_attr, batch,
           W1_0, b1_0, W2_0, b2_0,
           W1_1, b1_1, W2_1, b2_1,
           W1_2, b1_2, W2_2, b2_2,
           W_fc, b_fc):
    del edge_attr
    src = edge_index[0]
    dst = edge_index[1]
    src_e = src.reshape(2 * _NTILES, 25, -1, _K)
    dst_e = dst.reshape(2 * _NTILES, 25, -1, _K)
    src_c = src.reshape(_NTILES, 25, -1, _K)
    dst_c = dst.reshape(_NTILES, 25, -1, _K)

    agg2 = _make_sc_agg("edge")(x, src_e, dst_e)
    h = _tc_mlp(x, agg2, "edge", W1_0, b1_0, W2_0, b2_0)

    for (w1, b1, w2, b2) in [(W1_1, b1_1, W2_1, b2_1), (W1_2, b1_2, W2_2, b2_2)]:
        h2 = h.reshape(2 * _N, 128)
        agg2 = _make_sc_agg("col")(h2, src_c, dst_c)
        h = _tc_mlp(h, agg2, "col", w1, b1, w2, b2)

    return _tc_pool_fc(batch, h, W_fc, b_fc)
